# baseline (speedup 1.0000x reference)
import jax
import jax.numpy as jnp
from jax import lax
from jax.experimental import pallas as pl
from jax.experimental.pallas import tpu as pltpu
from jax.experimental.pallas import tpu_sc as plsc

B = 1024
L = 200
D = 128
NC = 2
NS = 16
NW = NC * NS
ROWS = B * L
BPW = ROWS // NW
CH = 200
NCHUNK = BPW // CH
HALF = CH // 2
NBUF = 4
LANES = 16


def _sc_body(idx_hbm, token_hbm, pos_hbm, out_hbm,
             idx_r, pos_v, buf0, buf1, buf2, buf3, gsem, ssem, isem):
    c_ax = lax.axis_index("c")
    s_ax = lax.axis_index("s")
    wid = s_ax * NC + c_ax
    bufs = (buf0, buf1, buf2, buf3)

    pltpu.sync_copy(pos_hbm, pos_v)

    def idx_copy(ci, slot, issue):
        cp = (pltpu.async_copy if issue else pltpu.make_async_copy)(
            idx_hbm.at[wid].at[pl.ds(2 * ci, 2)],
            idx_r.at[pl.ds(2 * slot, 2)], isem.at[slot])
        return cp

    def gather_pair(ci, b, issue):
        f = pltpu.async_copy if issue else pltpu.make_async_copy
        c0 = f(token_hbm.at[idx_r.at[2 * b]],
               bufs[b].at[pl.ds(0, HALF)], gsem.at[b])
        c1 = f(token_hbm.at[idx_r.at[2 * b + 1]],
               bufs[b].at[pl.ds(HALF, HALF)], gsem.at[b])
        return c0, c1

    def store(ci, b, issue):
        base = wid * BPW + ci * CH
        return (pltpu.async_copy if issue else pltpu.make_async_copy)(
            bufs[b], out_hbm.at[pl.ds(base, CH)], ssem.at[b])

    for k in range(NBUF):
        idx_copy(k, k, True)
    for k in range(2):
        idx_copy(k, k, False).wait()
        gather_pair(k, k, True)

    def do_chunk(ci, b):
        nb = (b + 2) % NBUF

        @pl.when(ci + 2 < NCHUNK)
        def _():
            idx_copy(ci + 2, nb, False).wait()

            @pl.when(ci >= 2)
            def _():
                store(ci - 2, nb, False).wait()

            gather_pair(ci + 2, nb, True)

        gather_pair(ci, b, False)[0].wait()
        gather_pair(ci, b, False)[1].wait()

        @pl.when(ci + NBUF < NCHUNK)
        def _():
            idx_copy(ci + NBUF, b, True)

        buf = bufs[b]

        @plsc.parallel_loop(0, CH, unroll=2)
        def _(r):
            for j in range(D // LANES):
                sl = pl.ds(j * LANES, LANES)
                buf[r, sl] = buf[r, sl] + pos_v[r, sl]

        store(ci, b, True)

    def group_body(g, carry):
        for b in range(NBUF):
            do_chunk(g * NBUF + b, b)
        return carry

    lax.fori_loop(0, NCHUNK // NBUF, group_body, 0)

    for ci in range(NCHUNK - NBUF, NCHUNK):
        store(ci, ci % NBUF, False).wait()


@jax.jit
def _embed(idx, token_table, pos_table):
    mesh = plsc.VectorSubcoreMesh(core_axis_name="c", subcore_axis_name="s")
    f = pl.kernel(
        _sc_body,
        out_type=jax.ShapeDtypeStruct((ROWS, D), jnp.float32),
        mesh=mesh,
        scratch_types=[
            pltpu.VMEM((NBUF * 2, HALF), jnp.int32),
            pltpu.VMEM((L, D), jnp.float32),
            pltpu.VMEM((CH, D), jnp.float32),
            pltpu.VMEM((CH, D), jnp.float32),
            pltpu.VMEM((CH, D), jnp.float32),
            pltpu.VMEM((CH, D), jnp.float32),
            pltpu.SemaphoreType.DMA((NBUF,)),
            pltpu.SemaphoreType.DMA((NBUF,)),
            pltpu.SemaphoreType.DMA((NBUF,)),
        ],
    )
    return f(idx, token_table, pos_table)


def kernel(inputs, token_table, pos_table):
    idx = inputs.reshape(NW, NCHUNK * 2, HALF).astype(jnp.int32)
    out = _embed(idx, token_table, pos_table)
    return out.reshape(B, L, D)


# speedup vs baseline: 1.0071x; 1.0071x over previous
import jax
import jax.numpy as jnp
from jax import lax
from jax.experimental import pallas as pl
from jax.experimental.pallas import tpu as pltpu
from jax.experimental.pallas import tpu_sc as plsc

B = 1024
L = 200
D = 128
NC = 2
NS = 16
NW = NC * NS
ROWS = B * L
BPW = ROWS // NW
CH = 200
NCHUNK = BPW // CH
HALF = CH // 2
NBUF = 3
LANES = 16


def _sc_body(idx_hbm, token_hbm, pos_hbm, out_hbm,
             idx_v, pos_v, buf0, buf1, buf2, gsem, ssem):
    c_ax = lax.axis_index("c")
    s_ax = lax.axis_index("s")
    wid = s_ax * NC + c_ax
    bufs = (buf0, buf1, buf2)

    pltpu.sync_copy(idx_hbm.at[wid], idx_v)
    pltpu.sync_copy(pos_hbm, pos_v)

    def gather_pair(ci, b):
        c0 = pltpu.async_copy(token_hbm.at[idx_v.at[2 * ci]],
                              bufs[b].at[pl.ds(0, HALF)], gsem.at[b])
        c1 = pltpu.async_copy(token_hbm.at[idx_v.at[2 * ci + 1]],
                              bufs[b].at[pl.ds(HALF, HALF)], gsem.at[b])
        return c0, c1

    def store(ci, b):
        base = wid * BPW + ci * CH
        return pltpu.async_copy(bufs[b], out_hbm.at[pl.ds(base, CH)],
                                ssem.at[b])

    def store_wait(ci, b):
        base = wid * BPW + ci * CH
        pltpu.make_async_copy(bufs[b], out_hbm.at[pl.ds(base, CH)],
                              ssem.at[b]).wait()

    def do_chunk(ci, b):
        nb = (b + 1) % NBUF

        @pl.when(jnp.logical_and(ci >= 2, ci + 1 < NCHUNK))
        def _():
            store_wait(ci - 2, nb)

        @pl.when(ci + 1 < NCHUNK)
        def _():
            gather_pair(ci + 1, nb)

        pltpu.make_async_copy(token_hbm.at[idx_v.at[2 * ci]],
                              bufs[b].at[pl.ds(0, HALF)], gsem.at[b]).wait()
        pltpu.make_async_copy(token_hbm.at[idx_v.at[2 * ci + 1]],
                              bufs[b].at[pl.ds(HALF, HALF)], gsem.at[b]).wait()

        buf = bufs[b]

        @plsc.parallel_loop(0, CH, unroll=4)
        def _(r):
            for j in range(D // LANES):
                sl = pl.ds(j * LANES, LANES)
                buf[r, sl] = buf[r, sl] + pos_v[r, sl]

        store(ci, b)

    gather_pair(0, 0)

    def group_body(g, carry):
        for b in range(NBUF):
            do_chunk(g * NBUF + b, b)
        return carry

    ngroups = NCHUNK // NBUF
    lax.fori_loop(0, ngroups, group_body, 0)
    for tail in range(ngroups * NBUF, NCHUNK):
        do_chunk(tail, tail % NBUF)

    for ci in range(NCHUNK - NBUF, NCHUNK):
        store_wait(ci, ci % NBUF)


@jax.jit
def _embed(idx, token_table, pos_table):
    mesh = plsc.VectorSubcoreMesh(core_axis_name="c", subcore_axis_name="s")
    f = pl.kernel(
        _sc_body,
        out_type=jax.ShapeDtypeStruct((ROWS, D), jnp.float32),
        mesh=mesh,
        scratch_types=[
            pltpu.VMEM((NCHUNK * 2, HALF), jnp.int32),
            pltpu.VMEM((CH, D), jnp.float32),
            pltpu.VMEM((CH, D), jnp.float32),
            pltpu.VMEM((CH, D), jnp.float32),
            pltpu.VMEM((CH, D), jnp.float32),
            pltpu.SemaphoreType.DMA((NBUF,)),
            pltpu.SemaphoreType.DMA((NBUF,)),
        ],
    )
    return f(idx, token_table, pos_table)


def kernel(inputs, token_table, pos_table):
    idx = inputs.reshape(NW, NCHUNK * 2, HALF).astype(jnp.int32)
    out = _embed(idx, token_table, pos_table)
    return out.reshape(B, L, D)
